# baseline (speedup 1.0000x reference)
import functools

import jax
import jax.numpy as jnp
from jax import lax
from jax.experimental import pallas as pl
from jax.experimental.pallas import tpu as pltpu
from jax.experimental.pallas import tpu_sc as plsc

_Q = 4.0
_M = 6
_TINY = float(jnp.finfo(jnp.float32).eps)


def _cround(x):
    xb = jax.lax.bitcast_convert_type(x, jnp.uint32)
    st = (xb & jnp.uint32(0x80000000)) | jnp.uint32(0x34000000)
    y = x - jax.lax.bitcast_convert_type(st, jnp.float32)
    return jnp.floor(y + 0.5)


def _is_even(s):
    return (s.astype(jnp.int32) & 1) == 0


def _digit_mod4(v):
    return (v.astype(jnp.int32) & 3).astype(jnp.float32)


def _g_x_parts(xs, fs):
    s = xs[0] - fs[0]
    best = jnp.abs(s)
    k = jnp.zeros_like(best)
    sk = s
    fk = fs[0]
    for i in range(1, 8):
        si = xs[i] - fs[i]
        d = jnp.abs(si)
        c = d > best
        best = jnp.where(c, d, best)
        k = jnp.where(c, float(i), k)
        sk = jnp.where(c, si, sk)
        fk = jnp.where(c, fs[i], fk)
    cond = (sk > 0.0) | ((sk == 0.0) & (fk < 0.0))
    nfk = fk + jnp.where(cond, 1.0, -1.0)
    return k, nfk


def _cpe8(xs):
    fs = [_cround(x) for x in xs]
    s0 = fs[0]
    for i in range(1, 8):
        s0 = s0 + fs[i]
    even0 = _is_even(s0)
    k0, nf0 = _g_x_parts(xs, fs)
    k0 = jnp.where(even0, -1.0, k0)
    y0 = [jnp.where(k0 == float(i), nf0, f) for i, f in enumerate(fs)]

    xs2 = [x - 0.5 for x in xs]
    fs2 = [_cround(x) for x in xs2]
    s1 = fs2[0]
    for i in range(1, 8):
        s1 = s1 + fs2[i]
    even1 = _is_even(s1)
    k1, nf1 = _g_x_parts(xs2, fs2)
    k1 = jnp.where(even1, -1.0, k1)
    y1 = [jnp.where(k1 == float(i), nf1, f) + 0.5 for i, f in enumerate(fs2)]

    d0 = (xs[0] - y0[0]) * (xs[0] - y0[0])
    d1 = (xs[0] - y1[0]) * (xs[0] - y1[0])
    for i in range(1, 8):
        d0 = d0 + (xs[i] - y0[i]) * (xs[i] - y0[i])
        d1 = d1 + (xs[i] - y1[i]) * (xs[i] - y1[i])
    c = d0 < d1
    return [jnp.where(c, a, b) for a, b in zip(y0, y1)]


def _encode_coords(xl):
    suf = [None] * 7
    suf[6] = xl[6]
    for j in range(5, 0, -1):
        suf[j] = xl[j] + suf[j + 1]
    c = [None] * 8
    c[0] = 0.5 * (xl[0] + suf[1]) - 3.5 * xl[7]
    for j in range(1, 7):
        c[j] = suf[j] - float(7 - j) * xl[7]
    c[7] = 2.0 * xl[7]
    return c


def _decode_Gb(b):
    h = 0.5 * b[7]
    Gb = [None] * 8
    Gb[0] = 2.0 * b[0] - b[1] + h
    for i in range(1, 6):
        Gb[i] = b[i] - b[i + 1] + h
    Gb[6] = b[6] + h
    Gb[7] = h
    return Gb


def _hnlq_body(beta_ref, eps_ref, x_ref, o_ref):
    beta = beta_ref[0]
    xs = [x_ref[i] for i in range(8)]
    t = [xs[i] / beta + eps_ref[i] for i in range(8)]
    xhat = None
    for layer in range(_M):
        xl = _cpe8(t)
        cc = _encode_coords(xl)
        b = [_digit_mod4(v) for v in cc]
        t = [v * 0.25 for v in xl]
        Gb = _decode_Gb(b)
        gq = _cpe8([v * 0.25 for v in Gb])
        xi = [g - _Q * q for g, q in zip(Gb, gq)]
        if layer == 0:
            xhat = xi
        else:
            w = float(_Q ** layer)
            xhat = [h + w * v for h, v in zip(xhat, xi)]
    for i in range(8):
        o_ref[i] = beta * xhat[i]


def _hnlq_transposed(xt, beta, eps, sb):
    s = xt.shape[1]
    grid = s // sb
    return pl.pallas_call(
        _hnlq_body,
        grid=(grid,),
        in_specs=[
            pl.BlockSpec(memory_space=pltpu.SMEM),
            pl.BlockSpec(memory_space=pltpu.SMEM),
            pl.BlockSpec((8, sb, 128), lambda i: (0, i, 0)),
        ],
        out_specs=pl.BlockSpec((8, sb, 128), lambda i: (0, i, 0)),
        out_shape=jax.ShapeDtypeStruct(xt.shape, jnp.float32),
    )(beta, eps, xt)


def _floor_sc(v):
    t = v.astype(jnp.int32).astype(jnp.float32)
    return t - jnp.where(v < t, 1.0, 0.0)


def _cround_sc(x):
    xb = lax.bitcast_convert_type(x, jnp.uint32)
    st = (xb & jnp.uint32(0x80000000)) | jnp.uint32(0x34000000)
    y = x - lax.bitcast_convert_type(st, jnp.float32)
    return _floor_sc(y + 0.5)


def _g_x_parts_sc(xs, fs):
    s = xs[0] - fs[0]
    best = jnp.abs(s)
    k = jnp.zeros_like(best)
    sk = s
    fk = fs[0]
    for i in range(1, 8):
        si = xs[i] - fs[i]
        d = jnp.abs(si)
        c = d > best
        best = jnp.where(c, d, best)
        k = jnp.where(c, float(i), k)
        sk = jnp.where(c, si, sk)
        fk = jnp.where(c, fs[i], fk)
    cond = (sk > 0.0) | ((sk == 0.0) & (fk < 0.0))
    nfk = fk + jnp.where(cond, 1.0, -1.0)
    return k, nfk


def _cpe8_sc(xs):
    fs = [_cround_sc(x) for x in xs]
    s0 = fs[0]
    for i in range(1, 8):
        s0 = s0 + fs[i]
    k0, nf0 = _g_x_parts_sc(xs, fs)
    k0 = jnp.where(_is_even(s0), -1.0, k0)
    y0 = [jnp.where(k0 == float(i), nf0, f) for i, f in enumerate(fs)]
    xs2 = [x - 0.5 for x in xs]
    fs2 = [_cround_sc(x) for x in xs2]
    s1 = fs2[0]
    for i in range(1, 8):
        s1 = s1 + fs2[i]
    k1, nf1 = _g_x_parts_sc(xs2, fs2)
    k1 = jnp.where(_is_even(s1), -1.0, k1)
    y1 = [jnp.where(k1 == float(i), nf1, f) + 0.5 for i, f in enumerate(fs2)]
    d0 = (xs[0] - y0[0]) * (xs[0] - y0[0])
    d1 = (xs[0] - y1[0]) * (xs[0] - y1[0])
    for i in range(1, 8):
        d0 = d0 + (xs[i] - y0[i]) * (xs[i] - y0[i])
        d1 = d1 + (xs[i] - y1[i]) * (xs[i] - y1[i])
    c = d0 < d1
    return [jnp.where(c, a, b) for a, b in zip(y0, y1)]


def _hnlq_point16(xs, bb, ebs):
    t = [xs[i] / bb + ebs[i] for i in range(8)]
    xhat = None
    for layer in range(_M):
        xl = _cpe8_sc(t)
        cc = _encode_coords(xl)
        b = [_digit_mod4(v) for v in cc]
        t = [v * 0.25 for v in xl]
        Gb = _decode_Gb(b)
        gq = _cpe8_sc([v * 0.25 for v in Gb])
        xi = [g - _Q * q for g, q in zip(Gb, gq)]
        if layer == 0:
            xhat = xi
        else:
            w = float(_Q ** layer)
            xhat = [h + w * v for h, v in zip(xhat, xi)]
    return [bb * h for h in xhat]


_SC_LANES = 16
_SC_WORKERS = 32
_SC_CHUNK = 2048


def _make_sc_kernel(n_points):
    span = n_points // _SC_WORKERS
    nch = span // _SC_CHUNK
    assert span % _SC_CHUNK == 0

    mesh = plsc.VectorSubcoreMesh(core_axis_name="c", subcore_axis_name="s")

    @functools.partial(
        pl.kernel,
        mesh=mesh,
        out_type=jax.ShapeDtypeStruct((8, n_points), jnp.float32),
        scratch_types=[
            pltpu.VMEM((8, _SC_CHUNK), jnp.float32),
            pltpu.VMEM((8, _SC_CHUNK), jnp.float32),
            pltpu.VMEM((8, _SC_LANES), jnp.float32),
            pltpu.VMEM((_SC_LANES,), jnp.float32),
        ],
    )
    def k(x_hbm, epsb_hbm, betab_hbm, out_hbm, xin, xout, epsv, betav):
        wid = lax.axis_index("s") * 2 + lax.axis_index("c")
        base = wid * span
        pltpu.sync_copy(epsb_hbm, epsv)
        pltpu.sync_copy(betab_hbm, betav)
        bb = betav[...]
        ebs = [epsv[i] for i in range(8)]

        def chunk_body(ci, carry):
            off = base + ci * _SC_CHUNK
            for i in range(8):
                pltpu.sync_copy(x_hbm.at[i, pl.ds(off, _SC_CHUNK)], xin.at[i])

            def pt_body(j, inner):
                sl = pl.ds(j * _SC_LANES, _SC_LANES)
                xs = [xin[i, sl] for i in range(8)]
                res = _hnlq_point16(xs, bb, ebs)
                for i in range(8):
                    xout[i, sl] = res[i]
                return inner

            lax.fori_loop(0, _SC_CHUNK // _SC_LANES, pt_body, 0)
            for i in range(8):
                pltpu.sync_copy(xout.at[i], out_hbm.at[i, pl.ds(off, _SC_CHUNK)])
            return carry

        lax.fori_loop(0, nch, chunk_body, 0)

    return k


def _hnlq_sc(xt, beta, eps):
    n = xt.shape[1]
    epsb = jnp.tile(eps[:, None], (1, _SC_LANES)).astype(jnp.float32)
    betab = jnp.full((_SC_LANES,), beta, dtype=jnp.float32)
    return _make_sc_kernel(n)(xt, epsb, betab)


def kernel(x, beta, alpha, G, G_inv, eps):
    n = x.shape[0]
    sb = 64
    n_sc = 3 * _SC_WORKERS * _SC_CHUNK
    n_tc = n - n_sc
    assert n_tc % (sb * 128) == 0
    xt = x.T
    out_sc = _hnlq_sc(xt[:, n_tc:], beta, eps)
    out_tc = _hnlq_transposed(
        xt[:, :n_tc].reshape(8, n_tc // 128, 128),
        jnp.reshape(beta, (1,)), eps, sb,
    ).reshape(8, n_tc)
    x_hat = jnp.concatenate([out_tc, out_sc], axis=1).T
    return x + jax.lax.stop_gradient(x_hat - x)


# speedup vs baseline: 1.6650x; 1.6650x over previous
import functools

import jax
import jax.numpy as jnp
from jax import lax
from jax.experimental import pallas as pl
from jax.experimental.pallas import tpu as pltpu
from jax.experimental.pallas import tpu_sc as plsc

_Q = 4.0
_M = 6
_TINY = float(jnp.finfo(jnp.float32).eps)


def _cround(x):
    xb = jax.lax.bitcast_convert_type(x, jnp.uint32)
    st = (xb & jnp.uint32(0x80000000)) | jnp.uint32(0x34000000)
    y = x - jax.lax.bitcast_convert_type(st, jnp.float32)
    return jnp.floor(y + 0.5)


def _is_even(s):
    return (s.astype(jnp.int32) & 1) == 0


def _digit_mod4(v):
    return (v.astype(jnp.int32) & 3).astype(jnp.float32)


def _g_x_parts(xs, fs):
    s = xs[0] - fs[0]
    best = jnp.abs(s)
    k = jnp.zeros_like(best)
    sk = s
    fk = fs[0]
    for i in range(1, 8):
        si = xs[i] - fs[i]
        d = jnp.abs(si)
        c = d > best
        best = jnp.where(c, d, best)
        k = jnp.where(c, float(i), k)
        sk = jnp.where(c, si, sk)
        fk = jnp.where(c, fs[i], fk)
    cond = (sk > 0.0) | ((sk == 0.0) & (fk < 0.0))
    nfk = fk + jnp.where(cond, 1.0, -1.0)
    return k, nfk


def _cpe8(xs):
    fs = [_cround(x) for x in xs]
    s0 = fs[0]
    for i in range(1, 8):
        s0 = s0 + fs[i]
    even0 = _is_even(s0)
    k0, nf0 = _g_x_parts(xs, fs)
    k0 = jnp.where(even0, -1.0, k0)
    y0 = [jnp.where(k0 == float(i), nf0, f) for i, f in enumerate(fs)]

    xs2 = [x - 0.5 for x in xs]
    fs2 = [_cround(x) for x in xs2]
    s1 = fs2[0]
    for i in range(1, 8):
        s1 = s1 + fs2[i]
    even1 = _is_even(s1)
    k1, nf1 = _g_x_parts(xs2, fs2)
    k1 = jnp.where(even1, -1.0, k1)
    y1 = [jnp.where(k1 == float(i), nf1, f) + 0.5 for i, f in enumerate(fs2)]

    d0 = (xs[0] - y0[0]) * (xs[0] - y0[0])
    d1 = (xs[0] - y1[0]) * (xs[0] - y1[0])
    for i in range(1, 8):
        d0 = d0 + (xs[i] - y0[i]) * (xs[i] - y0[i])
        d1 = d1 + (xs[i] - y1[i]) * (xs[i] - y1[i])
    c = d0 < d1
    return [jnp.where(c, a, b) for a, b in zip(y0, y1)]


def _encode_coords(xl):
    suf = [None] * 7
    suf[6] = xl[6]
    for j in range(5, 0, -1):
        suf[j] = xl[j] + suf[j + 1]
    c = [None] * 8
    c[0] = 0.5 * (xl[0] + suf[1]) - 3.5 * xl[7]
    for j in range(1, 7):
        c[j] = suf[j] - float(7 - j) * xl[7]
    c[7] = 2.0 * xl[7]
    return c


def _decode_Gb(b):
    h = 0.5 * b[7]
    Gb = [None] * 8
    Gb[0] = 2.0 * b[0] - b[1] + h
    for i in range(1, 6):
        Gb[i] = b[i] - b[i + 1] + h
    Gb[6] = b[6] + h
    Gb[7] = h
    return Gb


def _hnlq_body(beta_ref, eps_ref, x_ref, o_ref):
    beta = beta_ref[0]
    xs = [x_ref[i] for i in range(8)]
    t = [xs[i] / beta + eps_ref[i] for i in range(8)]
    xhat = None
    for layer in range(_M):
        xl = _cpe8(t)
        cc = _encode_coords(xl)
        b = [_digit_mod4(v) for v in cc]
        t = [v * 0.25 for v in xl]
        Gb = _decode_Gb(b)
        gq = _cpe8([v * 0.25 for v in Gb])
        xi = [g - _Q * q for g, q in zip(Gb, gq)]
        if layer == 0:
            xhat = xi
        else:
            w = float(_Q ** layer)
            xhat = [h + w * v for h, v in zip(xhat, xi)]
    for i in range(8):
        o_ref[i] = beta * xhat[i]


def _hnlq_transposed(xt, beta, eps, sb):
    s = xt.shape[1]
    grid = s // sb
    return pl.pallas_call(
        _hnlq_body,
        grid=(grid,),
        in_specs=[
            pl.BlockSpec(memory_space=pltpu.SMEM),
            pl.BlockSpec(memory_space=pltpu.SMEM),
            pl.BlockSpec((8, sb, 128), lambda i: (0, i, 0)),
        ],
        out_specs=pl.BlockSpec((8, sb, 128), lambda i: (0, i, 0)),
        out_shape=jax.ShapeDtypeStruct(xt.shape, jnp.float32),
    )(beta, eps, xt)


def _floor_sc(v):
    t = v.astype(jnp.int32).astype(jnp.float32)
    return t - jnp.where(v < t, 1.0, 0.0)


def _cround_sc(x):
    xb = lax.bitcast_convert_type(x, jnp.uint32)
    st = (xb & jnp.uint32(0x80000000)) | jnp.uint32(0x34000000)
    y = x - lax.bitcast_convert_type(st, jnp.float32)
    return _floor_sc(y + 0.5)


def _g_x_parts_sc(xs, fs):
    s = xs[0] - fs[0]
    best = jnp.abs(s)
    k = jnp.zeros_like(best)
    sk = s
    fk = fs[0]
    for i in range(1, 8):
        si = xs[i] - fs[i]
        d = jnp.abs(si)
        c = d > best
        best = jnp.where(c, d, best)
        k = jnp.where(c, float(i), k)
        sk = jnp.where(c, si, sk)
        fk = jnp.where(c, fs[i], fk)
    cond = (sk > 0.0) | ((sk == 0.0) & (fk < 0.0))
    nfk = fk + jnp.where(cond, 1.0, -1.0)
    return k, nfk


def _cpe8_sc(xs):
    fs = [_cround_sc(x) for x in xs]
    s0 = fs[0]
    for i in range(1, 8):
        s0 = s0 + fs[i]
    k0, nf0 = _g_x_parts_sc(xs, fs)
    k0 = jnp.where(_is_even(s0), -1.0, k0)
    y0 = [jnp.where(k0 == float(i), nf0, f) for i, f in enumerate(fs)]
    xs2 = [x - 0.5 for x in xs]
    fs2 = [_cround_sc(x) for x in xs2]
    s1 = fs2[0]
    for i in range(1, 8):
        s1 = s1 + fs2[i]
    k1, nf1 = _g_x_parts_sc(xs2, fs2)
    k1 = jnp.where(_is_even(s1), -1.0, k1)
    y1 = [jnp.where(k1 == float(i), nf1, f) + 0.5 for i, f in enumerate(fs2)]
    d0 = (xs[0] - y0[0]) * (xs[0] - y0[0])
    d1 = (xs[0] - y1[0]) * (xs[0] - y1[0])
    for i in range(1, 8):
        d0 = d0 + (xs[i] - y0[i]) * (xs[i] - y0[i])
        d1 = d1 + (xs[i] - y1[i]) * (xs[i] - y1[i])
    c = d0 < d1
    return [jnp.where(c, a, b) for a, b in zip(y0, y1)]


def _hnlq_point16(xs, bb, ebs):
    t = [xs[i] / bb + ebs[i] for i in range(8)]
    xhat = None
    for layer in range(_M):
        xl = _cpe8_sc(t)
        cc = _encode_coords(xl)
        b = [_digit_mod4(v) for v in cc]
        t = [v * 0.25 for v in xl]
        Gb = _decode_Gb(b)
        gq = _cpe8_sc([v * 0.25 for v in Gb])
        xi = [g - _Q * q for g, q in zip(Gb, gq)]
        if layer == 0:
            xhat = xi
        else:
            w = float(_Q ** layer)
            xhat = [h + w * v for h, v in zip(xhat, xi)]
    return [bb * h for h in xhat]


_SC_LANES = 16
_SC_WORKERS = 32
_SC_CHUNK = 512


def _make_sc_kernel(n_points):
    span = n_points // _SC_WORKERS
    nch = span // _SC_CHUNK
    assert span % _SC_CHUNK == 0

    mesh = plsc.VectorSubcoreMesh(core_axis_name="c", subcore_axis_name="s")

    @functools.partial(
        pl.kernel,
        mesh=mesh,
        out_type=jax.ShapeDtypeStruct((8, n_points), jnp.float32),
        scratch_types=[
            pltpu.VMEM((8, _SC_CHUNK), jnp.float32),
            pltpu.VMEM((8, _SC_CHUNK), jnp.float32),
            pltpu.VMEM((8, _SC_LANES), jnp.float32),
            pltpu.VMEM((_SC_LANES,), jnp.float32),
        ],
    )
    def k(x_hbm, epsb_hbm, betab_hbm, out_hbm, xin, xout, epsv, betav):
        wid = lax.axis_index("s") * 2 + lax.axis_index("c")
        base = wid * span
        pltpu.sync_copy(epsb_hbm, epsv)
        pltpu.sync_copy(betab_hbm, betav)
        bb = betav[...]
        ebs = [epsv[i] for i in range(8)]

        def chunk_body(ci, carry):
            off = base + ci * _SC_CHUNK
            for i in range(8):
                pltpu.sync_copy(x_hbm.at[i, pl.ds(off, _SC_CHUNK)], xin.at[i])

            def pt_body(j, inner):
                sl = pl.ds(j * _SC_LANES, _SC_LANES)
                xs = [xin[i, sl] for i in range(8)]
                res = _hnlq_point16(xs, bb, ebs)
                for i in range(8):
                    xout[i, sl] = res[i]
                return inner

            lax.fori_loop(0, _SC_CHUNK // _SC_LANES, pt_body, 0)
            for i in range(8):
                pltpu.sync_copy(xout.at[i], out_hbm.at[i, pl.ds(off, _SC_CHUNK)])
            return carry

        lax.fori_loop(0, nch, chunk_body, 0)

    return k


def _hnlq_sc(xt, beta, eps):
    n = xt.shape[1]
    epsb = jnp.tile(eps[:, None], (1, _SC_LANES)).astype(jnp.float32)
    betab = jnp.full((_SC_LANES,), beta, dtype=jnp.float32)
    return _make_sc_kernel(n)(xt, epsb, betab)


def kernel(x, beta, alpha, G, G_inv, eps):
    n = x.shape[0]
    sb = 64
    n_sc = _SC_WORKERS * _SC_CHUNK
    n_tc = n - n_sc
    assert n_tc % (sb * 128) == 0
    xt = x.T
    out_sc = _hnlq_sc(xt[:, n_tc:], beta, eps)
    out_tc = _hnlq_transposed(
        xt[:, :n_tc].reshape(8, n_tc // 128, 128),
        jnp.reshape(beta, (1,)), eps, sb,
    ).reshape(8, n_tc)
    x_hat = jnp.concatenate([out_tc, out_sc], axis=1).T
    return x + jax.lax.stop_gradient(x_hat - x)


# speedup vs baseline: 1.7162x; 1.0307x over previous
import functools

import jax
import jax.numpy as jnp
from jax import lax
from jax.experimental import pallas as pl
from jax.experimental.pallas import tpu as pltpu
from jax.experimental.pallas import tpu_sc as plsc

_Q = 4.0
_M = 6
_TINY = float(jnp.finfo(jnp.float32).eps)


def _cround(x):
    xb = jax.lax.bitcast_convert_type(x, jnp.uint32)
    st = (xb & jnp.uint32(0x80000000)) | jnp.uint32(0x34000000)
    y = x - jax.lax.bitcast_convert_type(st, jnp.float32)
    return jnp.floor(y + 0.5)


def _is_even(s):
    return (s.astype(jnp.int32) & 1) == 0


def _digit_mod4(v):
    return (v.astype(jnp.int32) & 3).astype(jnp.float32)


def _g_x_parts(xs, fs):
    s = xs[0] - fs[0]
    best = jnp.abs(s)
    k = jnp.zeros_like(best)
    sk = s
    fk = fs[0]
    for i in range(1, 8):
        si = xs[i] - fs[i]
        d = jnp.abs(si)
        c = d > best
        best = jnp.where(c, d, best)
        k = jnp.where(c, float(i), k)
        sk = jnp.where(c, si, sk)
        fk = jnp.where(c, fs[i], fk)
    cond = (sk > 0.0) | ((sk == 0.0) & (fk < 0.0))
    nfk = fk + jnp.where(cond, 1.0, -1.0)
    return k, nfk


def _cpe8(xs):
    fs = [_cround(x) for x in xs]
    s0 = fs[0]
    for i in range(1, 8):
        s0 = s0 + fs[i]
    even0 = _is_even(s0)
    k0, nf0 = _g_x_parts(xs, fs)
    k0 = jnp.where(even0, -1.0, k0)
    y0 = [jnp.where(k0 == float(i), nf0, f) for i, f in enumerate(fs)]

    xs2 = [x - 0.5 for x in xs]
    fs2 = [_cround(x) for x in xs2]
    s1 = fs2[0]
    for i in range(1, 8):
        s1 = s1 + fs2[i]
    even1 = _is_even(s1)
    k1, nf1 = _g_x_parts(xs2, fs2)
    k1 = jnp.where(even1, -1.0, k1)
    y1 = [jnp.where(k1 == float(i), nf1, f) + 0.5 for i, f in enumerate(fs2)]

    d0 = (xs[0] - y0[0]) * (xs[0] - y0[0])
    d1 = (xs[0] - y1[0]) * (xs[0] - y1[0])
    for i in range(1, 8):
        d0 = d0 + (xs[i] - y0[i]) * (xs[i] - y0[i])
        d1 = d1 + (xs[i] - y1[i]) * (xs[i] - y1[i])
    c = d0 < d1
    return [jnp.where(c, a, b) for a, b in zip(y0, y1)]


def _encode_coords(xl):
    suf = [None] * 7
    suf[6] = xl[6]
    for j in range(5, 0, -1):
        suf[j] = xl[j] + suf[j + 1]
    c = [None] * 8
    c[0] = 0.5 * (xl[0] + suf[1]) - 3.5 * xl[7]
    for j in range(1, 7):
        c[j] = suf[j] - float(7 - j) * xl[7]
    c[7] = 2.0 * xl[7]
    return c


def _decode_Gb(b):
    h = 0.5 * b[7]
    Gb = [None] * 8
    Gb[0] = 2.0 * b[0] - b[1] + h
    for i in range(1, 6):
        Gb[i] = b[i] - b[i + 1] + h
    Gb[6] = b[6] + h
    Gb[7] = h
    return Gb


def _hnlq_body(beta_ref, eps_ref, x_ref, o_ref):
    beta = beta_ref[0]
    xs = [x_ref[i] for i in range(8)]
    t = [xs[i] / beta + eps_ref[i] for i in range(8)]
    xhat = None
    for layer in range(_M):
        xl = _cpe8(t)
        cc = _encode_coords(xl)
        b = [_digit_mod4(v) for v in cc]
        t = [v * 0.25 for v in xl]
        Gb = _decode_Gb(b)
        gq = _cpe8([v * 0.25 for v in Gb])
        xi = [g - _Q * q for g, q in zip(Gb, gq)]
        if layer == 0:
            xhat = xi
        else:
            w = float(_Q ** layer)
            xhat = [h + w * v for h, v in zip(xhat, xi)]
    for i in range(8):
        o_ref[i] = beta * xhat[i]


def _hnlq_transposed(xt, beta, eps, sb):
    s = xt.shape[1]
    grid = s // sb
    return pl.pallas_call(
        _hnlq_body,
        grid=(grid,),
        in_specs=[
            pl.BlockSpec(memory_space=pltpu.SMEM),
            pl.BlockSpec(memory_space=pltpu.SMEM),
            pl.BlockSpec((8, sb, 128), lambda i: (0, i, 0)),
        ],
        out_specs=pl.BlockSpec((8, sb, 128), lambda i: (0, i, 0)),
        out_shape=jax.ShapeDtypeStruct(xt.shape, jnp.float32),
    )(beta, eps, xt)


def _floor_sc(v):
    t = v.astype(jnp.int32).astype(jnp.float32)
    return t - jnp.where(v < t, 1.0, 0.0)


def _cround_sc(x):
    xb = lax.bitcast_convert_type(x, jnp.uint32)
    st = (xb & jnp.uint32(0x80000000)) | jnp.uint32(0x34000000)
    y = x - lax.bitcast_convert_type(st, jnp.float32)
    return _floor_sc(y + 0.5)


def _g_x_parts_sc(xs, fs):
    s = xs[0] - fs[0]
    best = jnp.abs(s)
    k = jnp.zeros_like(best)
    sk = s
    fk = fs[0]
    for i in range(1, 8):
        si = xs[i] - fs[i]
        d = jnp.abs(si)
        c = d > best
        best = jnp.where(c, d, best)
        k = jnp.where(c, float(i), k)
        sk = jnp.where(c, si, sk)
        fk = jnp.where(c, fs[i], fk)
    cond = (sk > 0.0) | ((sk == 0.0) & (fk < 0.0))
    nfk = fk + jnp.where(cond, 1.0, -1.0)
    return k, nfk


def _cpe8_sc(xs):
    fs = [_cround_sc(x) for x in xs]
    s0 = fs[0]
    for i in range(1, 8):
        s0 = s0 + fs[i]
    k0, nf0 = _g_x_parts_sc(xs, fs)
    k0 = jnp.where(_is_even(s0), -1.0, k0)
    y0 = [jnp.where(k0 == float(i), nf0, f) for i, f in enumerate(fs)]
    xs2 = [x - 0.5 for x in xs]
    fs2 = [_cround_sc(x) for x in xs2]
    s1 = fs2[0]
    for i in range(1, 8):
        s1 = s1 + fs2[i]
    k1, nf1 = _g_x_parts_sc(xs2, fs2)
    k1 = jnp.where(_is_even(s1), -1.0, k1)
    y1 = [jnp.where(k1 == float(i), nf1, f) + 0.5 for i, f in enumerate(fs2)]
    d0 = (xs[0] - y0[0]) * (xs[0] - y0[0])
    d1 = (xs[0] - y1[0]) * (xs[0] - y1[0])
    for i in range(1, 8):
        d0 = d0 + (xs[i] - y0[i]) * (xs[i] - y0[i])
        d1 = d1 + (xs[i] - y1[i]) * (xs[i] - y1[i])
    c = d0 < d1
    return [jnp.where(c, a, b) for a, b in zip(y0, y1)]


def _hnlq_point16(xs, bb, ebs):
    t = [xs[i] / bb + ebs[i] for i in range(8)]
    xhat = None
    for layer in range(_M):
        xl = _cpe8_sc(t)
        cc = _encode_coords(xl)
        b = [_digit_mod4(v) for v in cc]
        t = [v * 0.25 for v in xl]
        Gb = _decode_Gb(b)
        gq = _cpe8_sc([v * 0.25 for v in Gb])
        xi = [g - _Q * q for g, q in zip(Gb, gq)]
        if layer == 0:
            xhat = xi
        else:
            w = float(_Q ** layer)
            xhat = [h + w * v for h, v in zip(xhat, xi)]
    return [bb * h for h in xhat]


_SC_LANES = 16
_SC_WORKERS = 32
_SC_CHUNK = 256


def _make_sc_kernel(n_points):
    span = n_points // _SC_WORKERS
    nch = span // _SC_CHUNK
    assert span % _SC_CHUNK == 0

    mesh = plsc.VectorSubcoreMesh(core_axis_name="c", subcore_axis_name="s")

    @functools.partial(
        pl.kernel,
        mesh=mesh,
        out_type=jax.ShapeDtypeStruct((8, n_points), jnp.float32),
        scratch_types=[
            pltpu.VMEM((8, _SC_CHUNK), jnp.float32),
            pltpu.VMEM((8, _SC_CHUNK), jnp.float32),
            pltpu.VMEM((8, _SC_LANES), jnp.float32),
            pltpu.VMEM((_SC_LANES,), jnp.float32),
        ],
    )
    def k(x_hbm, epsb_hbm, betab_hbm, out_hbm, xin, xout, epsv, betav):
        wid = lax.axis_index("s") * 2 + lax.axis_index("c")
        base = wid * span
        pltpu.sync_copy(epsb_hbm, epsv)
        pltpu.sync_copy(betab_hbm, betav)
        bb = betav[...]
        ebs = [epsv[i] for i in range(8)]

        def chunk_body(ci, carry):
            off = base + ci * _SC_CHUNK
            for i in range(8):
                pltpu.sync_copy(x_hbm.at[i, pl.ds(off, _SC_CHUNK)], xin.at[i])

            def pt_body(j, inner):
                sl = pl.ds(j * _SC_LANES, _SC_LANES)
                xs = [xin[i, sl] for i in range(8)]
                res = _hnlq_point16(xs, bb, ebs)
                for i in range(8):
                    xout[i, sl] = res[i]
                return inner

            lax.fori_loop(0, _SC_CHUNK // _SC_LANES, pt_body, 0)
            for i in range(8):
                pltpu.sync_copy(xout.at[i], out_hbm.at[i, pl.ds(off, _SC_CHUNK)])
            return carry

        lax.fori_loop(0, nch, chunk_body, 0)

    return k


def _hnlq_sc(xt, beta, eps):
    n = xt.shape[1]
    epsb = jnp.tile(eps[:, None], (1, _SC_LANES)).astype(jnp.float32)
    betab = jnp.full((_SC_LANES,), beta, dtype=jnp.float32)
    return _make_sc_kernel(n)(xt, epsb, betab)


def kernel(x, beta, alpha, G, G_inv, eps):
    n = x.shape[0]
    sb = 64
    n_sc = _SC_WORKERS * _SC_CHUNK
    n_tc = n - n_sc
    assert n_tc % (sb * 128) == 0
    xt = x.T
    out_sc = _hnlq_sc(xt[:, n_tc:], beta, eps)
    out_tc = _hnlq_transposed(
        xt[:, :n_tc].reshape(8, n_tc // 128, 128),
        jnp.reshape(beta, (1,)), eps, sb,
    ).reshape(8, n_tc)
    x_hat = jnp.concatenate([out_tc, out_sc], axis=1).T
    return x + jax.lax.stop_gradient(x_hat - x)


# speedup vs baseline: 1.7338x; 1.0103x over previous
import functools

import jax
import jax.numpy as jnp
from jax import lax
from jax.experimental import pallas as pl
from jax.experimental.pallas import tpu as pltpu
from jax.experimental.pallas import tpu_sc as plsc

_Q = 4.0
_M = 6
_TINY = float(jnp.finfo(jnp.float32).eps)


def _cround(x):
    xb = jax.lax.bitcast_convert_type(x, jnp.uint32)
    st = (xb & jnp.uint32(0x80000000)) | jnp.uint32(0x34000000)
    y = x - jax.lax.bitcast_convert_type(st, jnp.float32)
    return jnp.floor(y + 0.5)


def _is_even(s):
    return (s.astype(jnp.int32) & 1) == 0


def _digit_mod4(v):
    return (v.astype(jnp.int32) & 3).astype(jnp.float32)


def _g_x_parts(xs, fs):
    s = xs[0] - fs[0]
    best = jnp.abs(s)
    k = jnp.zeros_like(best)
    sk = s
    fk = fs[0]
    for i in range(1, 8):
        si = xs[i] - fs[i]
        d = jnp.abs(si)
        c = d > best
        best = jnp.where(c, d, best)
        k = jnp.where(c, float(i), k)
        sk = jnp.where(c, si, sk)
        fk = jnp.where(c, fs[i], fk)
    cond = (sk > 0.0) | ((sk == 0.0) & (fk < 0.0))
    nfk = fk + jnp.where(cond, 1.0, -1.0)
    return k, nfk


def _cpe8(xs):
    fs = [_cround(x) for x in xs]
    s0 = fs[0]
    for i in range(1, 8):
        s0 = s0 + fs[i]
    even0 = _is_even(s0)
    k0, nf0 = _g_x_parts(xs, fs)
    k0 = jnp.where(even0, -1.0, k0)
    y0 = [jnp.where(k0 == float(i), nf0, f) for i, f in enumerate(fs)]

    xs2 = [x - 0.5 for x in xs]
    fs2 = [_cround(x) for x in xs2]
    s1 = fs2[0]
    for i in range(1, 8):
        s1 = s1 + fs2[i]
    even1 = _is_even(s1)
    k1, nf1 = _g_x_parts(xs2, fs2)
    k1 = jnp.where(even1, -1.0, k1)
    y1 = [jnp.where(k1 == float(i), nf1, f) + 0.5 for i, f in enumerate(fs2)]

    d0 = _dist8(xs, y0)
    d1 = _dist8(xs, y1)
    c = d0 < d1
    return [jnp.where(c, a, b) for a, b in zip(y0, y1)]


def _dist8(xs, ys):
    t = [(x - y) * (x - y) for x, y in zip(xs, ys)]
    p = [t[i] + t[i + 4] for i in range(4)]
    q = [p[0] + p[2], p[1] + p[3]]
    return q[0] + q[1]


def _encode_coords(xl):
    suf = [None] * 7
    suf[6] = xl[6]
    for j in range(5, 0, -1):
        suf[j] = xl[j] + suf[j + 1]
    c = [None] * 8
    c[0] = 0.5 * (xl[0] + suf[1]) - 3.5 * xl[7]
    for j in range(1, 7):
        c[j] = suf[j] - float(7 - j) * xl[7]
    c[7] = 2.0 * xl[7]
    return c


def _decode_Gb(b):
    h = 0.5 * b[7]
    Gb = [None] * 8
    Gb[0] = 2.0 * b[0] - b[1] + h
    for i in range(1, 6):
        Gb[i] = b[i] - b[i + 1] + h
    Gb[6] = b[6] + h
    Gb[7] = h
    return Gb


def _hnlq_body(beta_ref, eps_ref, x_ref, o_ref):
    beta = beta_ref[0]
    xs = [x_ref[i] for i in range(8)]
    t = [xs[i] / beta + eps_ref[i] for i in range(8)]
    xhat = None
    for layer in range(_M):
        xl = _cpe8(t)
        cc = _encode_coords(xl)
        b = [_digit_mod4(v) for v in cc]
        t = [v * 0.25 for v in xl]
        Gb = _decode_Gb(b)
        gq = _cpe8([v * 0.25 for v in Gb])
        xi = [g - _Q * q for g, q in zip(Gb, gq)]
        if layer == 0:
            xhat = xi
        else:
            w = float(_Q ** layer)
            xhat = [h + w * v for h, v in zip(xhat, xi)]
    for i in range(8):
        o_ref[i] = beta * xhat[i]


def _hnlq_transposed(xt, beta, eps, sb):
    s = xt.shape[1]
    grid = s // sb
    return pl.pallas_call(
        _hnlq_body,
        grid=(grid,),
        in_specs=[
            pl.BlockSpec(memory_space=pltpu.SMEM),
            pl.BlockSpec(memory_space=pltpu.SMEM),
            pl.BlockSpec((8, sb, 128), lambda i: (0, i, 0)),
        ],
        out_specs=pl.BlockSpec((8, sb, 128), lambda i: (0, i, 0)),
        out_shape=jax.ShapeDtypeStruct(xt.shape, jnp.float32),
    )(beta, eps, xt)


def _floor_sc(v):
    t = v.astype(jnp.int32).astype(jnp.float32)
    return t - jnp.where(v < t, 1.0, 0.0)


def _cround_sc(x):
    xb = lax.bitcast_convert_type(x, jnp.uint32)
    st = (xb & jnp.uint32(0x80000000)) | jnp.uint32(0x34000000)
    y = x - lax.bitcast_convert_type(st, jnp.float32)
    return _floor_sc(y + 0.5)


def _g_x_parts_sc(xs, fs):
    s = xs[0] - fs[0]
    best = jnp.abs(s)
    k = jnp.zeros_like(best)
    sk = s
    fk = fs[0]
    for i in range(1, 8):
        si = xs[i] - fs[i]
        d = jnp.abs(si)
        c = d > best
        best = jnp.where(c, d, best)
        k = jnp.where(c, float(i), k)
        sk = jnp.where(c, si, sk)
        fk = jnp.where(c, fs[i], fk)
    cond = (sk > 0.0) | ((sk == 0.0) & (fk < 0.0))
    nfk = fk + jnp.where(cond, 1.0, -1.0)
    return k, nfk


def _cpe8_sc(xs):
    fs = [_cround_sc(x) for x in xs]
    s0 = fs[0]
    for i in range(1, 8):
        s0 = s0 + fs[i]
    k0, nf0 = _g_x_parts_sc(xs, fs)
    k0 = jnp.where(_is_even(s0), -1.0, k0)
    y0 = [jnp.where(k0 == float(i), nf0, f) for i, f in enumerate(fs)]
    xs2 = [x - 0.5 for x in xs]
    fs2 = [_cround_sc(x) for x in xs2]
    s1 = fs2[0]
    for i in range(1, 8):
        s1 = s1 + fs2[i]
    k1, nf1 = _g_x_parts_sc(xs2, fs2)
    k1 = jnp.where(_is_even(s1), -1.0, k1)
    y1 = [jnp.where(k1 == float(i), nf1, f) + 0.5 for i, f in enumerate(fs2)]
    d0 = _dist8(xs, y0)
    d1 = _dist8(xs, y1)
    c = d0 < d1
    return [jnp.where(c, a, b) for a, b in zip(y0, y1)]


def _hnlq_point16(xs, bb, ebs):
    t = [xs[i] / bb + ebs[i] for i in range(8)]
    xhat = None
    for layer in range(_M):
        xl = _cpe8_sc(t)
        cc = _encode_coords(xl)
        b = [_digit_mod4(v) for v in cc]
        t = [v * 0.25 for v in xl]
        Gb = _decode_Gb(b)
        gq = _cpe8_sc([v * 0.25 for v in Gb])
        xi = [g - _Q * q for g, q in zip(Gb, gq)]
        if layer == 0:
            xhat = xi
        else:
            w = float(_Q ** layer)
            xhat = [h + w * v for h, v in zip(xhat, xi)]
    return [bb * h for h in xhat]


_SC_LANES = 16
_SC_WORKERS = 32
_SC_CHUNK = 256


def _make_sc_kernel(n_points):
    span = n_points // _SC_WORKERS
    nch = span // _SC_CHUNK
    assert span % _SC_CHUNK == 0

    mesh = plsc.VectorSubcoreMesh(core_axis_name="c", subcore_axis_name="s")

    @functools.partial(
        pl.kernel,
        mesh=mesh,
        out_type=jax.ShapeDtypeStruct((8, n_points), jnp.float32),
        scratch_types=[
            pltpu.VMEM((8, _SC_CHUNK), jnp.float32),
            pltpu.VMEM((8, _SC_CHUNK), jnp.float32),
            pltpu.VMEM((8, _SC_LANES), jnp.float32),
            pltpu.VMEM((_SC_LANES,), jnp.float32),
        ],
    )
    def k(x_hbm, epsb_hbm, betab_hbm, out_hbm, xin, xout, epsv, betav):
        wid = lax.axis_index("s") * 2 + lax.axis_index("c")
        base = wid * span
        pltpu.sync_copy(epsb_hbm, epsv)
        pltpu.sync_copy(betab_hbm, betav)
        bb = betav[...]
        ebs = [epsv[i] for i in range(8)]

        def chunk_body(ci, carry):
            off = base + ci * _SC_CHUNK
            for i in range(8):
                pltpu.sync_copy(x_hbm.at[i, pl.ds(off, _SC_CHUNK)], xin.at[i])

            def pt_body(j, inner):
                sl = pl.ds(j * _SC_LANES, _SC_LANES)
                xs = [xin[i, sl] for i in range(8)]
                res = _hnlq_point16(xs, bb, ebs)
                for i in range(8):
                    xout[i, sl] = res[i]
                return inner

            lax.fori_loop(0, _SC_CHUNK // _SC_LANES, pt_body, 0)
            for i in range(8):
                pltpu.sync_copy(xout.at[i], out_hbm.at[i, pl.ds(off, _SC_CHUNK)])
            return carry

        lax.fori_loop(0, nch, chunk_body, 0)

    return k


def _hnlq_sc(xt, beta, eps):
    n = xt.shape[1]
    epsb = jnp.tile(eps[:, None], (1, _SC_LANES)).astype(jnp.float32)
    betab = jnp.full((_SC_LANES,), beta, dtype=jnp.float32)
    return _make_sc_kernel(n)(xt, epsb, betab)


def kernel(x, beta, alpha, G, G_inv, eps):
    n = x.shape[0]
    sb = 64
    n_sc = _SC_WORKERS * _SC_CHUNK
    n_tc = n - n_sc
    assert n_tc % (sb * 128) == 0
    xt = x.T
    out_sc = _hnlq_sc(xt[:, n_tc:], beta, eps)
    out_tc = _hnlq_transposed(
        xt[:, :n_tc].reshape(8, n_tc // 128, 128),
        jnp.reshape(beta, (1,)), eps, sb,
    ).reshape(8, n_tc)
    x_hat = jnp.concatenate([out_tc, out_sc], axis=1).T
    return x + jax.lax.stop_gradient(x_hat - x)
